# baseline (speedup 1.0000x reference)
import functools

import jax
import jax.numpy as jnp
from jax import lax
from jax.experimental import pallas as pl
from jax.experimental.pallas import tpu as pltpu
from jax.experimental.pallas import tpu_sc as plsc

BATCH = 4096
HIST = 50
D = 32
VOCAB = 1000000
NC, NS = 2, 16
NW = NC * NS
CTILE = 128
NCT = BATCH // CTILE


def _make_kernel():
    mesh = plsc.VectorSubcoreMesh(core_axis_name="c", subcore_axis_name="s")

    @functools.partial(
        pl.kernel,
        mesh=mesh,
        out_type=jax.ShapeDtypeStruct((HIST, 4, NCT, 8, CTILE), jnp.float32),
        scratch_types=[
            pltpu.VMEM((HIST, CTILE), jnp.int32),
            pltpu.VMEM((4, CTILE, D), jnp.float32),
            pltpu.VMEM((4, 4, 8, CTILE), jnp.float32),
            [pltpu.SemaphoreType.DMA] * 4,
            [pltpu.SemaphoreType.DMA] * 4,
        ],
        compiler_params=pltpu.CompilerParams(
            use_tc_tiling_on_sc=False, needs_layout_passes=False
        ),
    )
    def k(idx_hbm, table_hbm, out_hbm, idx_v, rows_v, out_v, gsem, osem):
        wid = lax.axis_index("s") * NC + lax.axis_index("c")
        pltpu.sync_copy(idx_hbm.at[:, wid], idx_v)

        def gather_start(h, slot):
            pltpu.async_copy(table_hbm.at[idx_v.at[h]], rows_v.at[slot], gsem[slot])

        def gather_wait(slot):
            pltpu.make_async_copy(
                table_hbm.at[idx_v.at[0]], rows_v.at[slot], gsem[slot]
            ).wait()

        def out_start(h, slot):
            pltpu.async_copy(out_v.at[slot], out_hbm.at[h, :, wid], osem[slot])

        def out_wait(h, slot):
            pltpu.make_async_copy(
                out_v.at[slot], out_hbm.at[h, :, wid], osem[slot]
            ).wait()

        def transpose_block(slot):
            lanes = lax.iota(jnp.int32, 16)
            rv = rows_v.at[slot]
            for r in range(4):
                vs = []
                for dlo in range(8):
                    d = 8 * r + dlo
                    dvec = jnp.full((16,), d, jnp.int32)
                    for kk in range(8):
                        blo = lanes + (16 * kk)
                        vs.append(plsc.load_gather(rv, [blo, dvec]))
                for dlo in range(8):
                    for kk in range(8):
                        out_v[slot, r, dlo, pl.ds(16 * kk, 16)] = vs[dlo * 8 + kk]

        def step(h, slot):
            @pl.when(h + 3 < HIST)
            def _():
                gather_start(h + 3, (slot + 3) % 4)

            gather_wait(slot)

            @pl.when(h >= 4)
            def _():
                out_wait(h - 4, slot)

            transpose_block(slot)
            out_start(h, slot)

        for p in range(3):
            gather_start(p, p)

        def body(i, carry):
            for p in range(4):
                step(4 * i + p, p)
            return carry

        lax.fori_loop(0, HIST // 4, body, 0)
        for p in range(2):
            step(HIST - 2 + p, (HIST - 2 + p) % 4)
        for h in range(HIST - 4, HIST):
            out_wait(h, h % 4)

    return k


_gather_kernel = _make_kernel()

_PAD_C = 2048


def _pad_body(in_ref, out_ref):
    x = in_ref[...]
    eye = jnp.eye(D, dtype=jnp.float32)
    y = jax.lax.dot_general(x, eye, (((0,), (0,)), ((), ())),
                            preferred_element_type=jnp.float32)
    out_ref[:, 0:D] = y


def _relayout(embT):
    return pl.pallas_call(
        _pad_body,
        grid=((VOCAB + _PAD_C - 1) // _PAD_C,),
        in_specs=[pl.BlockSpec((D, _PAD_C), lambda j: (0, j))],
        out_specs=pl.BlockSpec((_PAD_C, 128), lambda j: (j, 0)),
        out_shape=jax.ShapeDtypeStruct((VOCAB, 128), jnp.float32),
    )(embT)


def kernel(inputs, embeddings):
    idx3 = (inputs.T.astype(jnp.int32) * 4).reshape(HIST * BATCH).reshape(HIST, NCT, CTILE)
    emb_pad = _relayout(embeddings.T)
    emb2 = emb_pad.reshape(4 * VOCAB, D)
    out5 = _gather_kernel(idx3, emb2)
    return out5.transpose(2, 4, 0, 1, 3).reshape(BATCH, HIST, D)


# speedup vs baseline: 1.0363x; 1.0363x over previous
import functools

import jax
import jax.numpy as jnp
from jax import lax
from jax.experimental import pallas as pl
from jax.experimental.pallas import tpu as pltpu
from jax.experimental.pallas import tpu_sc as plsc

BATCH = 4096
HIST = 50
D = 32
VOCAB = 1000000
NC, NS = 2, 16
NW = NC * NS
CTILE = 128
NCT = BATCH // CTILE


def _make_kernel():
    mesh = plsc.VectorSubcoreMesh(core_axis_name="c", subcore_axis_name="s")

    @functools.partial(
        pl.kernel,
        mesh=mesh,
        out_type=jax.ShapeDtypeStruct((HIST, 4, NCT, 8, CTILE), jnp.float32),
        scratch_types=[
            pltpu.VMEM((HIST, CTILE), jnp.int32),
            pltpu.VMEM((4, CTILE, D), jnp.float32),
            pltpu.VMEM((4, 4, 8, CTILE), jnp.float32),
            [pltpu.SemaphoreType.DMA] * 4,
            [pltpu.SemaphoreType.DMA] * 4,
        ],
        compiler_params=pltpu.CompilerParams(
            use_tc_tiling_on_sc=False, needs_layout_passes=False
        ),
    )
    def k(idx_hbm, table_hbm, out_hbm, idx_v, rows_v, out_v, gsem, osem):
        wid = lax.axis_index("s") * NC + lax.axis_index("c")
        pltpu.sync_copy(idx_hbm.at[:, wid], idx_v)

        def gather_start(h, slot):
            pltpu.async_copy(table_hbm.at[idx_v.at[h]], rows_v.at[slot], gsem[slot])

        def gather_wait(slot):
            pltpu.make_async_copy(
                table_hbm.at[idx_v.at[0]], rows_v.at[slot], gsem[slot]
            ).wait()

        def out_start(h, slot):
            pltpu.async_copy(out_v.at[slot], out_hbm.at[h, :, wid], osem[slot])

        def out_wait(h, slot):
            pltpu.make_async_copy(
                out_v.at[slot], out_hbm.at[h, :, wid], osem[slot]
            ).wait()

        def transpose_block(slot):
            lanes = lax.iota(jnp.int32, 16)
            rv = rows_v.at[slot]
            for r in range(4):
                vs = []
                for dlo in range(8):
                    d = 8 * r + dlo
                    dvec = jnp.full((16,), d, jnp.int32)
                    for kk in range(8):
                        blo = lanes + (16 * kk)
                        vs.append(plsc.load_gather(rv, [blo, dvec]))
                for dlo in range(8):
                    for kk in range(8):
                        out_v[slot, r, dlo, pl.ds(16 * kk, 16)] = vs[dlo * 8 + kk]

        def step(h, slot):
            @pl.when(h + 3 < HIST)
            def _():
                gather_start(h + 3, (slot + 3) % 4)

            gather_wait(slot)

            @pl.when(h >= 4)
            def _():
                out_wait(h - 4, slot)

            transpose_block(slot)
            out_start(h, slot)

        for p in range(3):
            gather_start(p, p)

        def body(i, carry):
            for p in range(4):
                step(4 * i + p, p)
            return carry

        lax.fori_loop(0, HIST // 4, body, 0)
        for p in range(2):
            step(HIST - 2 + p, (HIST - 2 + p) % 4)
        for h in range(HIST - 4, HIST):
            out_wait(h, h % 4)

    return k


_gather_kernel = _make_kernel()

_PAD_C = 2048


def _pad_body(in_ref, out_ref):
    x = in_ref[...]
    out_ref[:, 0:D] = jnp.swapaxes(x, 0, 1)


def _relayout(embT):
    return pl.pallas_call(
        _pad_body,
        grid=((VOCAB + _PAD_C - 1) // _PAD_C,),
        in_specs=[pl.BlockSpec((D, _PAD_C), lambda j: (0, j))],
        out_specs=pl.BlockSpec((_PAD_C, 128), lambda j: (j, 0)),
        out_shape=jax.ShapeDtypeStruct((VOCAB, 128), jnp.float32),
    )(embT)


def kernel(inputs, embeddings):
    idx3 = (inputs.T.astype(jnp.int32) * 4).reshape(HIST * BATCH).reshape(HIST, NCT, CTILE)
    emb_pad = _relayout(embeddings.T)
    emb2 = emb_pad.reshape(4 * VOCAB, D)
    out5 = _gather_kernel(idx3, emb2)
    return out5.transpose(2, 4, 0, 1, 3).reshape(BATCH, HIST, D)


# speedup vs baseline: 1.5643x; 1.5095x over previous
import functools

import jax
import jax.numpy as jnp
from jax import lax
from jax.experimental import pallas as pl
from jax.experimental.pallas import tpu as pltpu
from jax.experimental.pallas import tpu_sc as plsc

BATCH = 4096
HIST = 50
D = 32
VOCAB = 1000000
NC, NS = 2, 16
NW = NC * NS
CTILE = 128
NCT = BATCH // CTILE


def _make_kernel():
    mesh = plsc.VectorSubcoreMesh(core_axis_name="c", subcore_axis_name="s")

    @functools.partial(
        pl.kernel,
        mesh=mesh,
        out_type=jax.ShapeDtypeStruct((HIST, 4, NCT, 8, CTILE), jnp.float32),
        scratch_types=[
            pltpu.VMEM((HIST, CTILE), jnp.int32),
            pltpu.VMEM((4, CTILE, D), jnp.float32),
            pltpu.VMEM((4, 4, 8, CTILE), jnp.float32),
            [pltpu.SemaphoreType.DMA] * 4,
            [pltpu.SemaphoreType.DMA] * 4,
        ],
        compiler_params=pltpu.CompilerParams(
            use_tc_tiling_on_sc=False, needs_layout_passes=False
        ),
    )
    def k(idx_hbm, table_hbm, out_hbm, idx_v, rows_v, out_v, gsem, osem):
        wid = lax.axis_index("s") * NC + lax.axis_index("c")
        pltpu.sync_copy(idx_hbm.at[:, wid], idx_v)

        def gather_start(h, slot):
            pltpu.async_copy(table_hbm.at[idx_v.at[h]], rows_v.at[slot], gsem[slot])

        def gather_wait(slot):
            pltpu.make_async_copy(
                table_hbm.at[idx_v.at[0]], rows_v.at[slot], gsem[slot]
            ).wait()

        def out_start(h, slot):
            pltpu.async_copy(out_v.at[slot], out_hbm.at[h, :, wid], osem[slot])

        def out_wait(h, slot):
            pltpu.make_async_copy(
                out_v.at[slot], out_hbm.at[h, :, wid], osem[slot]
            ).wait()

        def transpose_block(slot):
            lanes = lax.iota(jnp.int32, 16)
            rv = rows_v.at[slot]
            for r in range(4):
                vs = []
                for dlo in range(8):
                    d = 8 * r + dlo
                    dvec = jnp.full((16,), d, jnp.int32)
                    for kk in range(8):
                        blo = lanes + (16 * kk)
                        vs.append(plsc.load_gather(rv, [blo, dvec]))
                for dlo in range(8):
                    for kk in range(8):
                        out_v[slot, r, dlo, pl.ds(16 * kk, 16)] = vs[dlo * 8 + kk]

        def step(h, slot):
            @pl.when(h + 3 < HIST)
            def _():
                gather_start(h + 3, (slot + 3) % 4)

            gather_wait(slot)

            @pl.when(h >= 4)
            def _():
                out_wait(h - 4, slot)

            transpose_block(slot)
            out_start(h, slot)

        for p in range(3):
            gather_start(p, p)

        def body(i, carry):
            for p in range(4):
                step(4 * i + p, p)
            return carry

        lax.fori_loop(0, HIST // 4, body, 0)
        for p in range(2):
            step(HIST - 2 + p, (HIST - 2 + p) % 4)
        for h in range(HIST - 4, HIST):
            out_wait(h, h % 4)

    return k


_gather_kernel = _make_kernel()

_PAD_C = 8192


def _pad_body(in_ref, out_ref):
    x = in_ref[...]
    out_ref[:, 0:D] = jnp.swapaxes(x, 0, 1)


def _relayout(embT):
    return pl.pallas_call(
        _pad_body,
        grid=((VOCAB + _PAD_C - 1) // _PAD_C,),
        in_specs=[pl.BlockSpec((D, _PAD_C), lambda j: (0, j))],
        out_specs=pl.BlockSpec((_PAD_C, 128), lambda j: (j, 0)),
        out_shape=jax.ShapeDtypeStruct((VOCAB, 128), jnp.float32),
    )(embT)


def kernel(inputs, embeddings):
    idx3 = (inputs.T.astype(jnp.int32) * 4).reshape(HIST * BATCH).reshape(HIST, NCT, CTILE)
    emb_pad = _relayout(embeddings.T)
    emb2 = emb_pad.reshape(4 * VOCAB, D)
    out5 = _gather_kernel(idx3, emb2)
    return out5.transpose(2, 4, 0, 1, 3).reshape(BATCH, HIST, D)


# speedup vs baseline: 1.7004x; 1.0870x over previous
import functools

import jax
import jax.numpy as jnp
from jax import lax
from jax.experimental import pallas as pl
from jax.experimental.pallas import tpu as pltpu
from jax.experimental.pallas import tpu_sc as plsc

BATCH = 4096
HIST = 50
D = 32
VOCAB = 1000000
NC, NS = 2, 16
NW = NC * NS
CTILE = 128
NCT = BATCH // CTILE


def _make_kernel():
    mesh = plsc.VectorSubcoreMesh(core_axis_name="c", subcore_axis_name="s")

    @functools.partial(
        pl.kernel,
        mesh=mesh,
        out_type=jax.ShapeDtypeStruct((HIST, 4, NCT, 8, CTILE), jnp.float32),
        scratch_types=[
            pltpu.VMEM((HIST, CTILE), jnp.int32),
            pltpu.VMEM((4, CTILE, D), jnp.float32),
            pltpu.VMEM((4, 4, 8, CTILE), jnp.float32),
            [pltpu.SemaphoreType.DMA] * 4,
            [pltpu.SemaphoreType.DMA] * 4,
        ],
        compiler_params=pltpu.CompilerParams(
            use_tc_tiling_on_sc=False, needs_layout_passes=False
        ),
    )
    def k(idx_hbm, table_hbm, out_hbm, idx_v, rows_v, out_v, gsem, osem):
        wid = lax.axis_index("s") * NC + lax.axis_index("c")
        pltpu.sync_copy(idx_hbm.at[:, wid], idx_v)

        def gather_start(h, slot):
            pltpu.async_copy(table_hbm.at[idx_v.at[h]], rows_v.at[slot], gsem[slot])

        def gather_wait(slot):
            pltpu.make_async_copy(
                table_hbm.at[idx_v.at[0]], rows_v.at[slot], gsem[slot]
            ).wait()

        def out_start(h, slot):
            pltpu.async_copy(out_v.at[slot], out_hbm.at[h, :, wid], osem[slot])

        def out_wait(h, slot):
            pltpu.make_async_copy(
                out_v.at[slot], out_hbm.at[h, :, wid], osem[slot]
            ).wait()

        def transpose_block(slot):
            lanes = lax.iota(jnp.int32, 16)
            rv = rows_v.at[slot]
            for r in range(4):
                vs = []
                for dlo in range(8):
                    d = 8 * r + dlo
                    dvec = jnp.full((16,), d, jnp.int32)
                    for kk in range(8):
                        blo = lanes + (16 * kk)
                        vs.append(plsc.load_gather(rv, [blo, dvec]))
                for dlo in range(8):
                    for kk in range(8):
                        out_v[slot, r, dlo, pl.ds(16 * kk, 16)] = vs[dlo * 8 + kk]

        def step(h, slot):
            @pl.when(h + 3 < HIST)
            def _():
                gather_start(h + 3, (slot + 3) % 4)

            gather_wait(slot)

            @pl.when(h >= 4)
            def _():
                out_wait(h - 4, slot)

            transpose_block(slot)
            out_start(h, slot)

        for p in range(3):
            gather_start(p, p)

        def body(i, carry):
            for p in range(4):
                step(4 * i + p, p)
            return carry

        lax.fori_loop(0, HIST // 4, body, 0)
        for p in range(2):
            step(HIST - 2 + p, (HIST - 2 + p) % 4)
        for h in range(HIST - 4, HIST):
            out_wait(h, h % 4)

    return k


_gather_kernel = _make_kernel()

_PAD_C = 16384


def _pad_body(in_ref, out_ref):
    x = in_ref[...]
    out_ref[:, 0:D] = jnp.swapaxes(x, 0, 1)


def _relayout(embT):
    return pl.pallas_call(
        _pad_body,
        grid=((VOCAB + _PAD_C - 1) // _PAD_C,),
        in_specs=[pl.BlockSpec((D, _PAD_C), lambda j: (0, j))],
        out_specs=pl.BlockSpec((_PAD_C, 128), lambda j: (j, 0)),
        out_shape=jax.ShapeDtypeStruct((VOCAB, 128), jnp.float32),
    )(embT)


def kernel(inputs, embeddings):
    idx3 = (inputs.T.astype(jnp.int32) * 4).reshape(HIST * BATCH).reshape(HIST, NCT, CTILE)
    emb_pad = _relayout(embeddings.T)
    emb2 = emb_pad.reshape(4 * VOCAB, D)
    out5 = _gather_kernel(idx3, emb2)
    return out5.transpose(2, 4, 0, 1, 3).reshape(BATCH, HIST, D)


# speedup vs baseline: 1.7269x; 1.0156x over previous
import functools

import jax
import jax.numpy as jnp
from jax import lax
from jax.experimental import pallas as pl
from jax.experimental.pallas import tpu as pltpu
from jax.experimental.pallas import tpu_sc as plsc

BATCH = 4096
HIST = 50
D = 32
VOCAB = 1000000
NC, NS = 2, 16
NW = NC * NS
CTILE = 128
NCT = BATCH // CTILE


def _make_kernel():
    mesh = plsc.VectorSubcoreMesh(core_axis_name="c", subcore_axis_name="s")

    @functools.partial(
        pl.kernel,
        mesh=mesh,
        out_type=jax.ShapeDtypeStruct((HIST, 4, NCT, 8, CTILE), jnp.float32),
        scratch_types=[
            pltpu.VMEM((HIST, CTILE), jnp.int32),
            pltpu.VMEM((4, CTILE, D), jnp.float32),
            pltpu.VMEM((4, 4, 8, CTILE), jnp.float32),
            [pltpu.SemaphoreType.DMA] * 4,
            [pltpu.SemaphoreType.DMA] * 4,
        ],
        compiler_params=pltpu.CompilerParams(
            use_tc_tiling_on_sc=False, needs_layout_passes=False
        ),
    )
    def k(idx_hbm, table_hbm, out_hbm, idx_v, rows_v, out_v, gsem, osem):
        wid = lax.axis_index("s") * NC + lax.axis_index("c")
        pltpu.sync_copy(idx_hbm.at[:, wid], idx_v)

        def gather_start(h, slot):
            pltpu.async_copy(table_hbm.at[idx_v.at[h]], rows_v.at[slot], gsem[slot])

        def gather_wait(slot):
            pltpu.make_async_copy(
                table_hbm.at[idx_v.at[0]], rows_v.at[slot], gsem[slot]
            ).wait()

        def out_start(h, slot):
            pltpu.async_copy(out_v.at[slot], out_hbm.at[h, :, wid], osem[slot])

        def out_wait(h, slot):
            pltpu.make_async_copy(
                out_v.at[slot], out_hbm.at[h, :, wid], osem[slot]
            ).wait()

        def transpose_block(slot):
            lanes = lax.iota(jnp.int32, 16)
            rv = rows_v.at[slot]
            for r in range(4):
                vs = []
                for dlo in range(8):
                    d = 8 * r + dlo
                    dvec = jnp.full((16,), d, jnp.int32)
                    for kk in range(8):
                        blo = lanes + (16 * kk)
                        vs.append(plsc.load_gather(rv, [blo, dvec]))
                for dlo in range(8):
                    for kk in range(8):
                        out_v[slot, r, dlo, pl.ds(16 * kk, 16)] = vs[dlo * 8 + kk]

        def step(h, slot):
            @pl.when(h + 3 < HIST)
            def _():
                gather_start(h + 3, (slot + 3) % 4)

            gather_wait(slot)

            @pl.when(h >= 4)
            def _():
                out_wait(h - 4, slot)

            transpose_block(slot)
            out_start(h, slot)

        for p in range(3):
            gather_start(p, p)

        def body(i, carry):
            for p in range(4):
                step(4 * i + p, p)
            return carry

        lax.fori_loop(0, HIST // 4, body, 0)
        for p in range(2):
            step(HIST - 2 + p, (HIST - 2 + p) % 4)
        for h in range(HIST - 4, HIST):
            out_wait(h, h % 4)

    return k


_gather_kernel = _make_kernel()

_PAD_C = 32768


def _pad_body(in_ref, out_ref):
    x = in_ref[...]
    out_ref[:, 0:D] = jnp.swapaxes(x, 0, 1)


def _relayout(embT):
    return pl.pallas_call(
        _pad_body,
        grid=((VOCAB + _PAD_C - 1) // _PAD_C,),
        in_specs=[pl.BlockSpec((D, _PAD_C), lambda j: (0, j))],
        out_specs=pl.BlockSpec((_PAD_C, 128), lambda j: (j, 0)),
        out_shape=jax.ShapeDtypeStruct((VOCAB, 128), jnp.float32),
    )(embT)


def kernel(inputs, embeddings):
    idx3 = (inputs.T.astype(jnp.int32) * 4).reshape(HIST * BATCH).reshape(HIST, NCT, CTILE)
    emb_pad = _relayout(embeddings.T)
    emb2 = emb_pad.reshape(4 * VOCAB, D)
    out5 = _gather_kernel(idx3, emb2)
    return out5.transpose(2, 4, 0, 1, 3).reshape(BATCH, HIST, D)
